# baseline (speedup 1.0000x reference)
import functools

import jax
import jax.numpy as jnp
from jax import lax
from jax.experimental import pallas as pl
from jax.experimental.pallas import tpu as pltpu
from jax.experimental.pallas import tpu_sc as plsc

EMB_DIM = 64
BATCH = 16384
NC, NS, LANES = 2, 16, 16
NW = NC * NS
BPW = BATCH // NW
NGRP = BPW // LANES
TCOL = 128
NSLOT = 7
AHEAD = 6


def _sc_scores(u, v, Ut, Vt):
    mesh = plsc.VectorSubcoreMesh(core_axis_name="c", subcore_axis_name="s")

    @functools.partial(
        pl.kernel,
        out_type=jax.ShapeDtypeStruct((BATCH,), jnp.float32),
        mesh=mesh,
        scratch_types=[
            pltpu.VMEM((BPW,), jnp.int32),
            pltpu.VMEM((BPW,), jnp.int32),
            pltpu.VMEM((NSLOT, EMB_DIM, TCOL), jnp.float32),
            pltpu.VMEM((NSLOT, EMB_DIM, TCOL), jnp.float32),
            pltpu.VMEM((BPW,), jnp.float32),
            pltpu.SemaphoreType.DMA((NSLOT,)),
        ],
        compiler_params=pltpu.CompilerParams(needs_layout_passes=False),
    )
    def scores_kernel(u_hbm, v_hbm, Ut_hbm, Vt_hbm, out_hbm,
                      idx_u, idx_v, ubuf, vbuf, scores, sem):
        wid = lax.axis_index("s") * NC + lax.axis_index("c")
        base = wid * BPW
        lane = lax.iota(jnp.int32, LANES)
        crow = lane

        pltpu.sync_copy(u_hbm.at[pl.ds(base, BPW)], idx_u)
        pltpu.sync_copy(v_hbm.at[pl.ds(base, BPW)], idx_v)

        def copies_for(j, k):
            gbase = (j // LANES) * LANES
            qu = pl.multiple_of(idx_u[pl.ds(gbase, LANES)][k] & ~(TCOL - 1),
                                TCOL)
            qv = pl.multiple_of(idx_v[pl.ds(gbase, LANES)][k] & ~(TCOL - 1),
                                TCOL)
            slot = j % NSLOT
            return [
                pltpu.make_async_copy(Ut_hbm.at[:, pl.ds(qu, TCOL)],
                                      ubuf.at[slot], sem.at[slot]),
                pltpu.make_async_copy(Vt_hbm.at[:, pl.ds(qv, TCOL)],
                                      vbuf.at[slot], sem.at[slot]),
            ]

        def hsum(x):
            for s in (1, 2, 4, 8):
                x = x + jnp.take_along_axis(x, lane ^ s, axis=0)
            return x

        for k in range(AHEAD):
            for cp in copies_for(k, k):
                cp.start()

        def group_body(g, _):
            ru = idx_u[pl.ds(g * LANES, LANES)]
            rv = idx_v[pl.ds(g * LANES, LANES)]
            vec = jnp.zeros((LANES,), jnp.float32)
            for k in range(LANES):
                j_fire = g * LANES + k + AHEAD

                @pl.when(j_fire < BPW)
                def _fire():
                    for cp in copies_for(j_fire, (k + AHEAD) % LANES):
                        cp.start()

                j = g * LANES + k
                slot = j % NSLOT
                for cp in copies_for(j, k):
                    cp.wait()
                lu = jnp.broadcast_to(ru[k] & (TCOL - 1), (LANES,))
                lv = jnp.broadcast_to(rv[k] & (TCOL - 1), (LANES,))
                acc = (plsc.load_gather(ubuf.at[slot], [crow, lu])
                       * plsc.load_gather(vbuf.at[slot], [crow, lv]))
                for c in range(1, EMB_DIM // LANES):
                    acc = acc + (
                        plsc.load_gather(ubuf.at[slot], [crow + c * LANES, lu])
                        * plsc.load_gather(vbuf.at[slot], [crow + c * LANES, lv]))
                vec = jnp.where(lane == k, hsum(acc), vec)
            scores[pl.ds(g * LANES, LANES)] = vec
            return _

        lax.fori_loop(0, NGRP, group_body, 0)
        pltpu.sync_copy(scores, out_hbm.at[pl.ds(base, BPW)])

    return scores_kernel(u, v, Ut, Vt)


def _tail_kernel(s_ref, o_ref):
    x = jnp.clip(s_ref[...], -10.0, 10.0)
    o_ref[0, 0] = jnp.sum(jnp.log1p(jnp.exp(-x))) * (1.0 / BATCH)


def _tc_tail(scores):
    out = pl.pallas_call(
        _tail_kernel,
        out_shape=jax.ShapeDtypeStruct((1, 1), jnp.float32),
        in_specs=[pl.BlockSpec(memory_space=pltpu.VMEM)],
        out_specs=pl.BlockSpec(memory_space=pltpu.SMEM),
    )(scores.reshape(128, BATCH // 128))
    return out[0, 0]


def kernel(u, v, U, V):
    u = u.astype(jnp.int32)
    v = v.astype(jnp.int32)
    scores = _sc_scores(u, v, U.T, V.T)
    return _tc_tail(scores)
